# baseline (speedup 1.0000x reference)
import numpy as np
import jax
import jax.numpy as jnp
from jax.experimental import pallas as pl

_P_T = 0.1
_P_C = 0.01
_C_SPAN = False
_MASK_T_SPAN = 6
_MASK_C_SPAN = 1
_B, _D, _H, _W = 16, 256, 64, 512


def _make_span_from_seeds(seeds, span, total):
    inds = []
    for seed in seeds:
        for i in range(int(seed), int(seed) + span):
            if i >= total:
                break
            if i not in inds:
                inds.append(int(i))
    return np.array(inds, dtype=np.int64)


def _make_mask(shape, p, span, rng, use_span):
    mask = np.zeros(shape, dtype=bool)
    for i in range(shape[0]):
        seeds = np.array([], dtype=np.int64)
        while len(seeds) == 0 and p > 0:
            seeds = np.nonzero(rng.random(shape[1]) < p)[0]
        if use_span:
            idx = _make_span_from_seeds(seeds, span, shape[1])
            if idx.size > 0:
                mask[i, idx] = True
        else:
            mask[i, seeds] = True
    return mask


_rng = np.random.default_rng(0)
_MASK_T_NP = _make_mask((_B, _W), _P_T, _MASK_T_SPAN, _rng, True)
_MASK_C_NP = _make_mask((_B, _H), _P_C, _MASK_C_SPAN, _rng, _C_SPAN)

_DBLK = 16


def _select_kernel(x_ref, mt_ref, mc_ref, tr_ref, cr_ref, o_ref):
    x = x_ref[...]
    mt = mt_ref[...][:, :, None, :]
    mc = mc_ref[...][:, :, :, None]
    tr = tr_ref[...][None, :, :, None]
    cr = cr_ref[...][None, :, :, None]
    o_ref[...] = jnp.where(mc != 0, cr, jnp.where(mt != 0, tr, x))


def kernel(x, t_mask_replacement, c_mask_replacement):
    mask_t = jnp.asarray(_MASK_T_NP)
    mask_c = jnp.asarray(_MASK_C_NP)
    mt_f = mask_t.astype(jnp.float32).reshape(_B, 1, _W)
    mc_f = mask_c.astype(jnp.float32).reshape(_B, 1, _H)
    tr = t_mask_replacement.reshape(_D, 1)
    cr = c_mask_replacement.reshape(_D, 1)

    grid = (_B, _D // _DBLK)
    out = pl.pallas_call(
        _select_kernel,
        grid=grid,
        in_specs=[
            pl.BlockSpec((1, _DBLK, _H, _W), lambda b, d: (b, d, 0, 0)),
            pl.BlockSpec((1, 1, _W), lambda b, d: (b, 0, 0)),
            pl.BlockSpec((1, 1, _H), lambda b, d: (b, 0, 0)),
            pl.BlockSpec((_DBLK, 1), lambda b, d: (d, 0)),
            pl.BlockSpec((_DBLK, 1), lambda b, d: (d, 0)),
        ],
        out_specs=pl.BlockSpec((1, _DBLK, _H, _W), lambda b, d: (b, d, 0, 0)),
        out_shape=jax.ShapeDtypeStruct((_B, _D, _H, _W), jnp.float32),
    )(x, mt_f, mc_f, tr, cr)

    return (out, x, mask_t, mask_c)


# speedup vs baseline: 1.0632x; 1.0632x over previous
import numpy as np
import jax
import jax.numpy as jnp
from jax.experimental import pallas as pl

_P_T = 0.1
_P_C = 0.01
_C_SPAN = False
_MASK_T_SPAN = 6
_MASK_C_SPAN = 1
_B, _D, _H, _W = 16, 256, 64, 512


def _make_span_from_seeds(seeds, span, total):
    inds = []
    for seed in seeds:
        for i in range(int(seed), int(seed) + span):
            if i >= total:
                break
            if i not in inds:
                inds.append(int(i))
    return np.array(inds, dtype=np.int64)


def _make_mask(shape, p, span, rng, use_span):
    mask = np.zeros(shape, dtype=bool)
    for i in range(shape[0]):
        seeds = np.array([], dtype=np.int64)
        while len(seeds) == 0 and p > 0:
            seeds = np.nonzero(rng.random(shape[1]) < p)[0]
        if use_span:
            idx = _make_span_from_seeds(seeds, span, shape[1])
            if idx.size > 0:
                mask[i, idx] = True
        else:
            mask[i, seeds] = True
    return mask


_rng = np.random.default_rng(0)
_MASK_T_NP = _make_mask((_B, _W), _P_T, _MASK_T_SPAN, _rng, True)
_MASK_C_NP = _make_mask((_B, _H), _P_C, _MASK_C_SPAN, _rng, _C_SPAN)

_DBLK = 64


def _select_kernel(x_ref, mt_ref, mc_ref, tr_ref, cr_ref, o_ref):
    x = x_ref[...]
    mt = mt_ref[...][:, :, None, :]
    mc = mc_ref[...][:, :, :, None]
    tr = tr_ref[...][None, :, :, None]
    cr = cr_ref[...][None, :, :, None]
    o_ref[...] = jnp.where(mc != 0, cr, jnp.where(mt != 0, tr, x))


def kernel(x, t_mask_replacement, c_mask_replacement):
    mask_t = jnp.asarray(_MASK_T_NP)
    mask_c = jnp.asarray(_MASK_C_NP)
    mt_f = mask_t.astype(jnp.float32).reshape(_B, 1, _W)
    mc_f = mask_c.astype(jnp.float32).reshape(_B, 1, _H)
    tr = t_mask_replacement.reshape(_D, 1)
    cr = c_mask_replacement.reshape(_D, 1)

    grid = (_B, _D // _DBLK)
    out = pl.pallas_call(
        _select_kernel,
        grid=grid,
        in_specs=[
            pl.BlockSpec((1, _DBLK, _H, _W), lambda b, d: (b, d, 0, 0)),
            pl.BlockSpec((1, 1, _W), lambda b, d: (b, 0, 0)),
            pl.BlockSpec((1, 1, _H), lambda b, d: (b, 0, 0)),
            pl.BlockSpec((_DBLK, 1), lambda b, d: (d, 0)),
            pl.BlockSpec((_DBLK, 1), lambda b, d: (d, 0)),
        ],
        out_specs=pl.BlockSpec((1, _DBLK, _H, _W), lambda b, d: (b, d, 0, 0)),
        out_shape=jax.ShapeDtypeStruct((_B, _D, _H, _W), jnp.float32),
    )(x, mt_f, mc_f, tr, cr)

    return (out, x, mask_t, mask_c)
